# baseline (speedup 1.0000x reference)
import functools

import jax
import jax.numpy as jnp
from jax import lax
from jax.experimental import pallas as pl
from jax.experimental.pallas import tpu as pltpu
from jax.experimental.pallas import tpu_sc as plsc

_N_VOCAB = 2048
_D = 1024
_B = 4
_SEQ = 2048
_NC = 2
_NS = 16
_NW = _NC * _NS
_S_PER_W = _SEQ // _NW
_CHUNK = 16
_LANES = 16


def _sc_embed(x_flat, table, pos):
    mesh = plsc.VectorSubcoreMesh(core_axis_name="c", subcore_axis_name="s")

    @functools.partial(
        pl.kernel,
        mesh=mesh,
        out_type=jax.ShapeDtypeStruct((_B * _SEQ, _D), jnp.float32),
        scratch_types=[
            pltpu.VMEM((_CHUNK,), jnp.int32),
            pltpu.VMEM((_CHUNK, _D), jnp.float32),
            pltpu.VMEM((_CHUNK, _D), jnp.float32),
            pltpu.SemaphoreType.DMA,
        ],
    )
    def k(x_hbm, tab_hbm, pos_hbm, out_hbm, idx_v, rows_v, pos_v, sem):
        wid = lax.axis_index("s") * _NC + lax.axis_index("c")
        s_base = wid * _S_PER_W

        def chunk_body(c, _):
            s0 = s_base + c * _CHUNK
            pltpu.sync_copy(pos_hbm.at[pl.ds(s0, _CHUNK)], pos_v)

            def batch_body(b, _):
                row0 = b * _SEQ + s0
                pltpu.sync_copy(x_hbm.at[pl.ds(row0, _CHUNK)], idx_v)
                pltpu.async_copy(tab_hbm.at[idx_v], rows_v, sem).wait()

                def row_body(r, _):
                    def vec_body(j, _):
                        o = j * (4 * _LANES)
                        for u in range(4):
                            sl = pl.ds(o + u * _LANES, _LANES)
                            rows_v[r, sl] = rows_v[r, sl] + pos_v[r, sl]
                        return 0

                    lax.fori_loop(0, _D // (4 * _LANES), vec_body, 0)
                    return 0

                lax.fori_loop(0, _CHUNK, row_body, 0)
                pltpu.sync_copy(rows_v, out_hbm.at[pl.ds(row0, _CHUNK)])
                return 0

            lax.fori_loop(0, _B, batch_body, 0)
            return 0

        lax.fori_loop(0, _S_PER_W // _CHUNK, chunk_body, 0)

    return k(x_flat, table, pos)


@jax.jit
def kernel(x, token_table, pos_emb):
    out_flat = _sc_embed(x.reshape(-1), token_table, pos_emb)
    return out_flat.reshape(_B, _SEQ, _D)
